# baseline (speedup 1.0000x reference)
import functools

import jax
import jax.numpy as jnp
from jax import lax
from jax.experimental import pallas as pl
from jax.experimental.pallas import tpu as pltpu
from jax.experimental.pallas import tpu_sc as plsc

NC = 2
NS = 16
L = 16
CB = 128


def _project_body(w_ref, e_ref, o0_ref, o1_ref):
    r = lax.dot_general(
        w_ref[...], e_ref[...], (((1,), (0,)), ((), ())),
        preferred_element_type=jnp.float32,
        precision=lax.Precision.HIGHEST)
    o0_ref[...] = r[0]
    o1_ref[...] = r[1]


def _project_table(Wp8, et):
    V = et.shape[1]
    C = 65536
    grid = pl.cdiv(V, C)
    return pl.pallas_call(
        _project_body,
        grid=(grid,),
        in_specs=[
            pl.BlockSpec((8, et.shape[0]), lambda i: (0, 0)),
            pl.BlockSpec((et.shape[0], C), lambda i: (0, i)),
        ],
        out_specs=[
            pl.BlockSpec((C,), lambda i: (i,)),
            pl.BlockSpec((C,), lambda i: (i,)),
        ],
        out_shape=[
            jax.ShapeDtypeStruct((V,), jnp.float32),
            jax.ShapeDtypeStruct((V,), jnp.float32),
        ],
    )(Wp8, et)


def _epilogue_body(a_ref, bias_ref, o_ref):
    z = a_ref[...] + bias_ref[...]
    s = jax.nn.sigmoid(z)
    m = jnp.max(s, axis=0, keepdims=True)
    lse = m + jnp.log(jnp.sum(jnp.exp(s - m), axis=0, keepdims=True))
    o_ref[...] = s - lse


def _epilogue(acc2, bias_col):
    B = acc2.shape[1]
    BLK = 4096
    return pl.pallas_call(
        _epilogue_body,
        grid=(B // BLK,),
        in_specs=[
            pl.BlockSpec((2, BLK), lambda i: (0, i)),
            pl.BlockSpec((2, 1), lambda i: (0, 0)),
        ],
        out_specs=pl.BlockSpec((2, BLK), lambda i: (0, i)),
        out_shape=jax.ShapeDtypeStruct((2, B), jnp.float32),
    )(acc2, bias_col)


def _make_sc_sum(S, B):
    NW = NC * NS
    BPW = B // NW
    NCHUNK = BPW // CB
    mesh = plsc.VectorSubcoreMesh(
        core_axis_name="c", subcore_axis_name="s",
        num_cores=NC, num_subcores=NS)

    NB = 8

    @functools.partial(
        pl.kernel,
        out_type=jax.ShapeDtypeStruct((2, B), jnp.float32),
        mesh=mesh,
        compiler_params=pltpu.CompilerParams(use_tc_tiling_on_sc=False),
        scratch_types=[
            pltpu.VMEM((S, BPW), jnp.int32),
            pltpu.VMEM((NB, CB), jnp.float32),
            pltpu.VMEM((NB, CB), jnp.float32),
            pltpu.VMEM((BPW,), jnp.float32),
            pltpu.VMEM((BPW,), jnp.float32),
            pltpu.SemaphoreType.DMA((NB,)),
            pltpu.SemaphoreType.DMA((NB,)),
        ],
    )
    def sc_sum(x_hbm, p0_hbm, p1_hbm, out_hbm,
               idx_v, b0_v, b1_v, a0_v, a1_v, sem0, sem1):
        wid = lax.axis_index("s") * NC + lax.axis_index("c")
        base = wid * BPW
        pltpu.sync_copy(x_hbm.at[:, pl.ds(base, BPW)], idx_v)

        zero = jnp.zeros((L,), jnp.float32)

        def zbody(i, carry):
            a0_v[pl.ds(i * L, L)] = zero
            a1_v[pl.ds(i * L, L)] = zero
            return carry
        lax.fori_loop(0, BPW // L, zbody, 0)

        T = NCHUNK * S

        def islice(t):
            c = t // S
            s = t - c * S
            return idx_v.at[s, pl.ds(c * CB, CB)]

        def start(t):
            slot = lax.rem(t, NB)
            isl = islice(t)
            pltpu.async_copy(p0_hbm.at[isl], b0_v.at[slot], sem0.at[slot])
            pltpu.async_copy(p1_hbm.at[isl], b1_v.at[slot], sem1.at[slot])

        def prime(t, carry):
            start(t)
            return carry
        lax.fori_loop(0, NB, prime, 0)

        def step(t, carry):
            slot = lax.rem(t, NB)
            isl = islice(t)
            pltpu.make_async_copy(p0_hbm.at[isl], b0_v.at[slot],
                                  sem0.at[slot]).wait()
            pltpu.make_async_copy(p1_hbm.at[isl], b1_v.at[slot],
                                  sem1.at[slot]).wait()
            rowbase = (t // S) * CB

            def addrow(r, carry2):
                plsc.addupdate(a0_v.at[pl.ds(rowbase + r * L, L)],
                               b0_v[slot, pl.ds(r * L, L)])
                plsc.addupdate(a1_v.at[pl.ds(rowbase + r * L, L)],
                               b1_v[slot, pl.ds(r * L, L)])
                return carry2
            lax.fori_loop(0, CB // L, addrow, 0)

            @pl.when(t + NB < T)
            def _():
                start(t + NB)
            return carry
        lax.fori_loop(0, T, step, 0)

        pltpu.sync_copy(a0_v, out_hbm.at[0, pl.ds(base, BPW)])
        pltpu.sync_copy(a1_v, out_hbm.at[1, pl.ds(base, BPW)])

    return sc_sum


def kernel(x, embed_table, W, b):
    S, B = x.shape
    V, D = embed_table.shape
    O = W.shape[0]
    Wp8 = jnp.zeros((8, D), jnp.float32).at[:O, :].set(W)
    p0, p1 = _project_table(Wp8, embed_table.T)
    acc2 = _make_sc_sum(S, B)(x.astype(jnp.int32), p0, p1)
    return _epilogue(acc2, b.reshape(O, 1)).T


# speedup vs baseline: 1.0010x; 1.0010x over previous
import functools

import jax
import jax.numpy as jnp
from jax import lax
from jax.experimental import pallas as pl
from jax.experimental.pallas import tpu as pltpu
from jax.experimental.pallas import tpu_sc as plsc

NC = 2
NS = 16
L = 16
CB = 128


def _project_body(w_ref, e_ref, o0_ref, o1_ref):
    r = lax.dot_general(
        w_ref[...], e_ref[...], (((1,), (0,)), ((), ())),
        preferred_element_type=jnp.float32,
        precision=lax.Precision.HIGHEST)
    o0_ref[...] = r[0]
    o1_ref[...] = r[1]


def _project_table(Wp8, et):
    V = et.shape[1]
    C = 32768
    grid = pl.cdiv(V, C)
    return pl.pallas_call(
        _project_body,
        grid=(grid,),
        in_specs=[
            pl.BlockSpec((8, et.shape[0]), lambda i: (0, 0)),
            pl.BlockSpec((et.shape[0], C), lambda i: (0, i)),
        ],
        out_specs=[
            pl.BlockSpec((C,), lambda i: (i,)),
            pl.BlockSpec((C,), lambda i: (i,)),
        ],
        out_shape=[
            jax.ShapeDtypeStruct((V,), jnp.float32),
            jax.ShapeDtypeStruct((V,), jnp.float32),
        ],
    )(Wp8, et)


def _epilogue_body(a_ref, bias_ref, o_ref):
    z = a_ref[...] + bias_ref[...]
    s = jax.nn.sigmoid(z)
    m = jnp.max(s, axis=0, keepdims=True)
    lse = m + jnp.log(jnp.sum(jnp.exp(s - m), axis=0, keepdims=True))
    o_ref[...] = s - lse


def _epilogue(acc2, bias_col):
    B = acc2.shape[1]
    BLK = 4096
    return pl.pallas_call(
        _epilogue_body,
        grid=(B // BLK,),
        in_specs=[
            pl.BlockSpec((2, BLK), lambda i: (0, i)),
            pl.BlockSpec((2, 1), lambda i: (0, 0)),
        ],
        out_specs=pl.BlockSpec((2, BLK), lambda i: (0, i)),
        out_shape=jax.ShapeDtypeStruct((2, B), jnp.float32),
    )(acc2, bias_col)


def _make_sc_sum(S, B):
    NW = NC * NS
    BPW = B // NW
    NCHUNK = BPW // CB
    mesh = plsc.VectorSubcoreMesh(
        core_axis_name="c", subcore_axis_name="s",
        num_cores=NC, num_subcores=NS)

    NB = 8

    @functools.partial(
        pl.kernel,
        out_type=jax.ShapeDtypeStruct((2, B), jnp.float32),
        mesh=mesh,
        compiler_params=pltpu.CompilerParams(use_tc_tiling_on_sc=False),
        scratch_types=[
            pltpu.VMEM((S, BPW), jnp.int32),
            pltpu.VMEM((NB, CB), jnp.float32),
            pltpu.VMEM((NB, CB), jnp.float32),
            pltpu.VMEM((BPW,), jnp.float32),
            pltpu.VMEM((BPW,), jnp.float32),
            pltpu.SemaphoreType.DMA((NB,)),
            pltpu.SemaphoreType.DMA((NB,)),
        ],
    )
    def sc_sum(x_hbm, p0_hbm, p1_hbm, out_hbm,
               idx_v, b0_v, b1_v, a0_v, a1_v, sem0, sem1):
        wid = lax.axis_index("s") * NC + lax.axis_index("c")
        base = wid * BPW
        pltpu.sync_copy(x_hbm.at[:, pl.ds(base, BPW)], idx_v)

        zero = jnp.zeros((L,), jnp.float32)

        def zbody(i, carry):
            a0_v[pl.ds(i * L, L)] = zero
            a1_v[pl.ds(i * L, L)] = zero
            return carry
        lax.fori_loop(0, BPW // L, zbody, 0)

        T = NCHUNK * S

        def islice(t):
            c = t // S
            s = t - c * S
            return idx_v.at[s, pl.ds(c * CB, CB)]

        def start(t):
            slot = lax.rem(t, NB)
            isl = islice(t)
            pltpu.async_copy(p0_hbm.at[isl], b0_v.at[slot], sem0.at[slot])
            pltpu.async_copy(p1_hbm.at[isl], b1_v.at[slot], sem1.at[slot])

        def prime(t, carry):
            start(t)
            return carry
        lax.fori_loop(0, NB, prime, 0)

        def step(t, carry):
            slot = lax.rem(t, NB)
            isl = islice(t)
            rowbase = (t // S) * CB
            pltpu.make_async_copy(p0_hbm.at[isl], b0_v.at[slot],
                                  sem0.at[slot]).wait()
            for r in range(CB // L):
                plsc.addupdate(a0_v.at[pl.ds(rowbase + r * L, L)],
                               b0_v[slot, pl.ds(r * L, L)])
            pltpu.make_async_copy(p1_hbm.at[isl], b1_v.at[slot],
                                  sem1.at[slot]).wait()
            for r in range(CB // L):
                plsc.addupdate(a1_v.at[pl.ds(rowbase + r * L, L)],
                               b1_v[slot, pl.ds(r * L, L)])

            @pl.when(t + NB < T)
            def _():
                start(t + NB)
            return carry
        lax.fori_loop(0, T, step, 0)

        pltpu.sync_copy(a0_v, out_hbm.at[0, pl.ds(base, BPW)])
        pltpu.sync_copy(a1_v, out_hbm.at[1, pl.ds(base, BPW)])

    return sc_sum


def kernel(x, embed_table, W, b):
    S, B = x.shape
    V, D = embed_table.shape
    O = W.shape[0]
    Wp8 = jnp.zeros((8, D), jnp.float32).at[:O, :].set(W)
    p0, p1 = _project_table(Wp8, embed_table.T)
    acc2 = _make_sc_sum(S, B)(x.astype(jnp.int32), p0, p1)
    return _epilogue(acc2, b.reshape(O, 1)).T


# speedup vs baseline: 1.0233x; 1.0222x over previous
import functools

import jax
import jax.numpy as jnp
from jax import lax
from jax.experimental import pallas as pl
from jax.experimental.pallas import tpu as pltpu
from jax.experimental.pallas import tpu_sc as plsc

NC = 2
NS = 16
L = 16
CB = 128


def _project_body(w_ref, e_ref, o0_ref, o1_ref):
    r = lax.dot_general(
        w_ref[...], e_ref[...], (((1,), (0,)), ((), ())),
        preferred_element_type=jnp.float32,
        precision=lax.Precision.HIGHEST)
    o0_ref[...] = r[0]
    o1_ref[...] = r[1]


def _project_table(Wp8, et):
    V = et.shape[1]
    C = 50176
    grid = pl.cdiv(V, C)
    return pl.pallas_call(
        _project_body,
        grid=(grid,),
        in_specs=[
            pl.BlockSpec((8, et.shape[0]), lambda i: (0, 0)),
            pl.BlockSpec((et.shape[0], C), lambda i: (0, i)),
        ],
        out_specs=[
            pl.BlockSpec((C,), lambda i: (i,)),
            pl.BlockSpec((C,), lambda i: (i,)),
        ],
        out_shape=[
            jax.ShapeDtypeStruct((V,), jnp.float32),
            jax.ShapeDtypeStruct((V,), jnp.float32),
        ],
    )(Wp8, et)


def _epilogue_body(a_ref, bias_ref, o_ref):
    z = a_ref[...] + bias_ref[...]
    s = jax.nn.sigmoid(z)
    m = jnp.max(s, axis=0, keepdims=True)
    lse = m + jnp.log(jnp.sum(jnp.exp(s - m), axis=0, keepdims=True))
    o_ref[...] = s - lse


def _epilogue(acc2, bias_col):
    B = acc2.shape[1]
    BLK = 4096
    return pl.pallas_call(
        _epilogue_body,
        grid=(B // BLK,),
        in_specs=[
            pl.BlockSpec((2, BLK), lambda i: (0, i)),
            pl.BlockSpec((2, 1), lambda i: (0, 0)),
        ],
        out_specs=pl.BlockSpec((2, BLK), lambda i: (0, i)),
        out_shape=jax.ShapeDtypeStruct((2, B), jnp.float32),
    )(acc2, bias_col)


def _make_sc_sum(S, B):
    NW = NC * NS
    BPW = B // NW
    NCHUNK = BPW // CB
    mesh = plsc.VectorSubcoreMesh(
        core_axis_name="c", subcore_axis_name="s",
        num_cores=NC, num_subcores=NS)

    NB = 8

    @functools.partial(
        pl.kernel,
        out_type=jax.ShapeDtypeStruct((2, B), jnp.float32),
        mesh=mesh,
        compiler_params=pltpu.CompilerParams(use_tc_tiling_on_sc=False),
        scratch_types=[
            pltpu.VMEM((S, BPW), jnp.int32),
            pltpu.VMEM((NB, CB), jnp.float32),
            pltpu.VMEM((NB, CB), jnp.float32),
            pltpu.VMEM((BPW,), jnp.float32),
            pltpu.VMEM((BPW,), jnp.float32),
            pltpu.SemaphoreType.DMA((NB,)),
            pltpu.SemaphoreType.DMA((NB,)),
        ],
    )
    def sc_sum(x_hbm, p0_hbm, p1_hbm, out_hbm,
               idx_v, b0_v, b1_v, a0_v, a1_v, sem0, sem1):
        wid = lax.axis_index("s") * NC + lax.axis_index("c")
        base = wid * BPW
        pltpu.sync_copy(x_hbm.at[:, pl.ds(base, BPW)], idx_v)

        zero = jnp.zeros((L,), jnp.float32)

        def zbody(i, carry):
            a0_v[pl.ds(i * L, L)] = zero
            a1_v[pl.ds(i * L, L)] = zero
            return carry
        lax.fori_loop(0, BPW // L, zbody, 0)

        T = NCHUNK * S

        def islice(t):
            c = t // S
            s = t - c * S
            return idx_v.at[s, pl.ds(c * CB, CB)]

        def start(t):
            slot = lax.rem(t, NB)
            isl = islice(t)
            pltpu.async_copy(p0_hbm.at[isl], b0_v.at[slot], sem0.at[slot])
            pltpu.async_copy(p1_hbm.at[isl], b1_v.at[slot], sem1.at[slot])

        def prime(t, carry):
            start(t)
            return carry
        lax.fori_loop(0, NB, prime, 0)

        def step(t, carry):
            slot = lax.rem(t, NB)
            isl = islice(t)
            rowbase = (t // S) * CB
            pltpu.make_async_copy(p0_hbm.at[isl], b0_v.at[slot],
                                  sem0.at[slot]).wait()
            for r in range(CB // L):
                plsc.addupdate(a0_v.at[pl.ds(rowbase + r * L, L)],
                               b0_v[slot, pl.ds(r * L, L)])
            pltpu.make_async_copy(p1_hbm.at[isl], b1_v.at[slot],
                                  sem1.at[slot]).wait()
            for r in range(CB // L):
                plsc.addupdate(a1_v.at[pl.ds(rowbase + r * L, L)],
                               b1_v[slot, pl.ds(r * L, L)])

            @pl.when(t + NB < T)
            def _():
                start(t + NB)
            return carry
        lax.fori_loop(0, T, step, 0)

        pltpu.sync_copy(a0_v, out_hbm.at[0, pl.ds(base, BPW)])
        pltpu.sync_copy(a1_v, out_hbm.at[1, pl.ds(base, BPW)])

    return sc_sum


def kernel(x, embed_table, W, b):
    S, B = x.shape
    V, D = embed_table.shape
    O = W.shape[0]
    Wp8 = jnp.zeros((8, D), jnp.float32).at[:O, :].set(W)
    p0, p1 = _project_table(Wp8, embed_table.T)
    acc2 = _make_sc_sum(S, B)(x.astype(jnp.int32), p0, p1)
    return _epilogue(acc2, b.reshape(O, 1)).T
